# baseline (speedup 1.0000x reference)
import functools

import jax
import jax.numpy as jnp
from jax import lax
from jax.experimental import pallas as pl
from jax.experimental.pallas import tpu as pltpu
from jax.experimental.pallas import tpu_sc as plsc

_N = 10000
_E = 320000
_EP = 327680
_C = 2
_EC = _EP // _C
_D = 128
_NSUB = 16
_NPAD = 10240
_RPS = _NPAD // _NSUB
_GW = 128
_SW = 128
_BE = 1280
_NBE = _EC // _BE
_BN = 400
_EPS = 1e-5


def _sc_gather(table, gidx_all, c):
    mesh = plsc.VectorSubcoreMesh(core_axis_name="c", subcore_axis_name="s")

    @functools.partial(
        pl.kernel,
        out_type=jax.ShapeDtypeStruct((2 * _EC, _D), jnp.float32),
        mesh=mesh,
    )
    def k(t_hbm, i_hbm, o_hbm):
        def body(i_vmem, o_vmem):
            pltpu.sync_copy(t_hbm.at[i_vmem.at[0]], o_vmem)

        pltpu.emit_pipeline(
            body,
            grid=(2 * _EC // _GW,),
            in_specs=[pl.BlockSpec((1, _GW), lambda i: (c, i))],
            out_specs=[pl.BlockSpec((_GW, _D), lambda i: (i, 0))],
            core_axis_name=("c", "s"),
            dimension_semantics=(pltpu.PARALLEL,),
        )(i_hbm, o_hbm)

    return k(table, gidx_all)


def _sc_segment_sum(delta, sidx_all, zeros_hbm, c):
    mesh = plsc.VectorSubcoreMesh(core_axis_name="c", subcore_axis_name="s")

    @functools.partial(
        pl.kernel,
        out_type=jax.ShapeDtypeStruct((2, _NPAD, _D), jnp.float32),
        mesh=mesh,
        scratch_types=[pltpu.VMEM_SHARED((_NPAD, _D), jnp.float32)],
    )
    def k(d_hbm, i_hbm, z_hbm, o_hbm, agg_sh):
        cid = lax.axis_index("c")
        sid = lax.axis_index("s")
        row0 = sid * _RPS
        pltpu.sync_copy(z_hbm.at[pl.ds(row0, _RPS)], agg_sh.at[pl.ds(row0, _RPS)])
        plsc.subcore_barrier()

        def body(d_vmem, i_vmem):
            pltpu.sync_copy(d_vmem, agg_sh.at[i_vmem.at[0]], add=True)

        pltpu.emit_pipeline(
            body,
            grid=(_EC // _SW,),
            in_specs=[
                pl.BlockSpec((_SW, _D), lambda i: (i, 0)),
                pl.BlockSpec((1, _SW), lambda i: (c, i)),
            ],
            out_specs=[],
            core_axis_name=("c", "s"),
            dimension_semantics=(pltpu.PARALLEL,),
        )(d_hbm, i_hbm)
        plsc.subcore_barrier()
        pltpu.sync_copy(agg_sh.at[pl.ds(row0, _RPS)],
                        o_hbm.at[cid].at[pl.ds(row0, _RPS)])

    return k(delta, sidx_all, zeros_hbm)


def _tc_node_proj(node2d, w23):
    def body(x_ref, w_ref, o_ref):
        x = x_ref[...]
        o_ref[0] = jnp.dot(x, w_ref[:_D], preferred_element_type=jnp.float32)
        o_ref[1] = jnp.dot(x, w_ref[_D:], preferred_element_type=jnp.float32)

    return pl.pallas_call(
        body,
        grid=(_N // _BN,),
        in_specs=[
            pl.BlockSpec((_BN, _D), lambda i: (i, 0)),
            pl.BlockSpec((2 * _D, _D), lambda i: (0, 0)),
        ],
        out_specs=pl.BlockSpec((2, _BN, _D), lambda i: (0, i, 0)),
        out_shape=jax.ShapeDtypeStruct((2, _N, _D), jnp.float32),
    )(node2d, w23)


def _tc_edge(edge2d, gpair, w1, g1, b1, c):
    def body(e_ref, g_ref, w_ref, g1_ref, b1_ref, oe_ref, od_ref):
        e = e_ref[...]
        h = jnp.dot(e, w_ref[...], preferred_element_type=jnp.float32)
        h = h + g_ref[0] + g_ref[1]
        t = jnp.tanh(h)
        mu = jnp.mean(t, axis=-1, keepdims=True)
        var = jnp.mean((t - mu) ** 2, axis=-1, keepdims=True)
        d = (t - mu) / jnp.sqrt(var + _EPS) * g1_ref[...] + b1_ref[...]
        od_ref[...] = d
        oe_ref[...] = e + d

    nblk = _NBE if c < _C - 1 else (_E - (_C - 1) * _EC) // _BE
    return pl.pallas_call(
        body,
        grid=(nblk,),
        in_specs=[
            pl.BlockSpec((_BE, _D), lambda i: (i + c * _NBE, 0)),
            pl.BlockSpec((2, _BE, _D), lambda i: (0, i, 0)),
            pl.BlockSpec((_D, _D), lambda i: (0, 0)),
            pl.BlockSpec((1, _D), lambda i: (0, 0)),
            pl.BlockSpec((1, _D), lambda i: (0, 0)),
        ],
        out_specs=[
            pl.BlockSpec((_BE, _D), lambda i: (i, 0)),
            pl.BlockSpec((_BE, _D), lambda i: (i, 0)),
        ],
        out_shape=[
            jax.ShapeDtypeStruct((_EC, _D), jnp.float32),
            jax.ShapeDtypeStruct((_EC, _D), jnp.float32),
        ],
    )(edge2d, gpair, w1, g1, b1)


def _tc_node(node2d, aggps, deg, w, g2, b2):
    def body(x_ref, *rest):
        a_refs = rest[:_C]
        d_ref, w_ref, g_ref, b_ref, o_ref = rest[_C:]
        x = x_ref[...]
        acc = a_refs[0][0] + a_refs[0][1]
        for a in a_refs[1:]:
            acc = acc + a[0] + a[1]
        agg = acc / d_ref[...]
        h = jnp.dot(x, w_ref[:_D], preferred_element_type=jnp.float32)
        h = h + jnp.dot(agg, w_ref[_D:], preferred_element_type=jnp.float32)
        t = jnp.tanh(h)
        mu = jnp.mean(t, axis=-1, keepdims=True)
        var = jnp.mean((t - mu) ** 2, axis=-1, keepdims=True)
        o_ref[...] = x + (t - mu) / jnp.sqrt(var + _EPS) * g_ref[...] + b_ref[...]

    return pl.pallas_call(
        body,
        grid=(_N // _BN,),
        in_specs=[pl.BlockSpec((_BN, _D), lambda i: (i, 0))]
        + [pl.BlockSpec((2, _BN, _D), lambda i: (0, i, 0)) for _ in range(_C)]
        + [
            pl.BlockSpec((_BN, 1), lambda i: (i, 0)),
            pl.BlockSpec((2 * _D, _D), lambda i: (0, 0)),
            pl.BlockSpec((1, _D), lambda i: (0, 0)),
            pl.BlockSpec((1, _D), lambda i: (0, 0)),
        ],
        out_specs=pl.BlockSpec((_BN, _D), lambda i: (i, 0)),
        out_shape=jax.ShapeDtypeStruct((_N, _D), jnp.float32),
    )(node2d, *aggps, deg, w, g2, b2)


def kernel(mesh_mesh_bond_embedding, mesh_node_embedding, edge_pairs,
           num_of_linked_nodes, W_n2e, g1, b1, W_e2n, g2, b2):
    node2d = mesh_node_embedding.reshape(_N, _D)
    edge2d = mesh_mesh_bond_embedding.reshape(_E, _D)
    src = edge_pairs[:, 0]
    dst = edge_pairs[:, 1]
    ipadn = _N + (jnp.arange(_EP - _E, dtype=jnp.int32) % (_NPAD - _N))
    srcp = jnp.concatenate([src, ipadn]).reshape(_C, _EC)
    dstp = jnp.concatenate([dst + _N, ipadn]).reshape(_C, _EC)
    gidx_all = jnp.concatenate([srcp, dstp], axis=1)
    zeros_hbm = jnp.zeros((_NPAD, _D), jnp.float32)

    proj = _tc_node_proj(node2d, W_n2e[_D:])
    table = proj.reshape(2 * _N, _D)
    w1 = W_n2e[:_D]
    g1r, b1r = g1.reshape(1, _D), b1.reshape(1, _D)

    gpairs = [_sc_gather(table, gidx_all, c).reshape(2, _EC, _D)
              for c in range(_C)]
    edge_outs, deltas = [], []
    for c in range(_C):
        edge_out_c, delta_c = _tc_edge(edge2d, gpairs[c], w1, g1r, b1r, c)
        edge_outs.append(edge_out_c)
        deltas.append(delta_c)
    aggps = [_sc_segment_sum(deltas[c], srcp, zeros_hbm, c) for c in range(_C)]

    node_out2 = _tc_node(node2d, aggps, num_of_linked_nodes, W_e2n,
                         g2.reshape(1, _D), b2.reshape(1, _D))
    edge_out2 = jnp.concatenate(edge_outs, axis=0)[:_E]
    return (edge_out2.reshape(1, _E, _D), node_out2.reshape(1, _N, _D))


# speedup vs baseline: 1.0189x; 1.0189x over previous
import functools

import jax
import jax.numpy as jnp
from jax import lax
from jax.experimental import pallas as pl
from jax.experimental.pallas import tpu as pltpu
from jax.experimental.pallas import tpu_sc as plsc

_N = 10000
_E = 320000
_EP = 327680
_C = 4
_EC = _EP // _C
_D = 128
_NSUB = 16
_NPAD = 10240
_RPS = _NPAD // _NSUB
_GW = 128
_SW = 128
_BE = 1280
_NBE = _EC // _BE
_BN = 400
_EPS = 1e-5


def _sc_gather(table, gidx_all, c):
    mesh = plsc.VectorSubcoreMesh(core_axis_name="c", subcore_axis_name="s")

    @functools.partial(
        pl.kernel,
        out_type=jax.ShapeDtypeStruct((2 * _EC, _D), jnp.float32),
        mesh=mesh,
    )
    def k(t_hbm, i_hbm, o_hbm):
        def body(i_vmem, o_vmem):
            pltpu.sync_copy(t_hbm.at[i_vmem.at[0]], o_vmem)

        pltpu.emit_pipeline(
            body,
            grid=(2 * _EC // _GW,),
            in_specs=[pl.BlockSpec((1, _GW), lambda i: (c, i))],
            out_specs=[pl.BlockSpec((_GW, _D), lambda i: (i, 0))],
            core_axis_name=("c", "s"),
            dimension_semantics=(pltpu.PARALLEL,),
        )(i_hbm, o_hbm)

    return k(table, gidx_all)


def _sc_segment_sum(deltas, sidx_all, zeros_hbm):
    mesh = plsc.VectorSubcoreMesh(core_axis_name="c", subcore_axis_name="s")

    @functools.partial(
        pl.kernel,
        out_type=jax.ShapeDtypeStruct((2, _NPAD, _D), jnp.float32),
        mesh=mesh,
        scratch_types=[pltpu.VMEM_SHARED((_NPAD, _D), jnp.float32)],
    )
    def k(*refs):
        d_hbms = refs[:_C]
        i_hbm, z_hbm, o_hbm, agg_sh = refs[_C:]
        cid = lax.axis_index("c")
        sid = lax.axis_index("s")
        row0 = sid * _RPS
        pltpu.sync_copy(z_hbm.at[pl.ds(row0, _RPS)], agg_sh.at[pl.ds(row0, _RPS)])
        plsc.subcore_barrier()

        def body(d_vmem, i_vmem):
            pltpu.sync_copy(d_vmem, agg_sh.at[i_vmem.at[0]], add=True)

        for c in range(_C):
            pltpu.emit_pipeline(
                body,
                grid=(_EC // _SW,),
                in_specs=[
                    pl.BlockSpec((_SW, _D), lambda i: (i, 0)),
                    pl.BlockSpec((1, _SW), lambda i, c=c: (c, i)),
                ],
                out_specs=[],
                core_axis_name=("c", "s"),
                dimension_semantics=(pltpu.PARALLEL,),
            )(d_hbms[c], i_hbm)
        plsc.subcore_barrier()
        pltpu.sync_copy(agg_sh.at[pl.ds(row0, _RPS)],
                        o_hbm.at[cid].at[pl.ds(row0, _RPS)])

    return k(*deltas, sidx_all, zeros_hbm)


def _tc_node_proj(node2d, w23):
    def body(x_ref, w_ref, o_ref):
        x = x_ref[...]
        o_ref[0] = jnp.dot(x, w_ref[:_D], preferred_element_type=jnp.float32)
        o_ref[1] = jnp.dot(x, w_ref[_D:], preferred_element_type=jnp.float32)

    return pl.pallas_call(
        body,
        grid=(_N // _BN,),
        in_specs=[
            pl.BlockSpec((_BN, _D), lambda i: (i, 0)),
            pl.BlockSpec((2 * _D, _D), lambda i: (0, 0)),
        ],
        out_specs=pl.BlockSpec((2, _BN, _D), lambda i: (0, i, 0)),
        out_shape=jax.ShapeDtypeStruct((2, _N, _D), jnp.float32),
    )(node2d, w23)


def _tc_edge(edge2d, gpair, w1, g1, b1, c):
    def body(e_ref, g_ref, w_ref, g1_ref, b1_ref, oe_ref, od_ref):
        e = e_ref[...]
        h = jnp.dot(e, w_ref[...], preferred_element_type=jnp.float32)
        h = h + g_ref[0] + g_ref[1]
        t = jnp.tanh(h)
        mu = jnp.mean(t, axis=-1, keepdims=True)
        var = jnp.mean((t - mu) ** 2, axis=-1, keepdims=True)
        d = (t - mu) / jnp.sqrt(var + _EPS) * g1_ref[...] + b1_ref[...]
        od_ref[...] = d
        oe_ref[...] = e + d

    nblk = _NBE if c < _C - 1 else (_E - (_C - 1) * _EC) // _BE
    return pl.pallas_call(
        body,
        grid=(nblk,),
        in_specs=[
            pl.BlockSpec((_BE, _D), lambda i: (i + c * _NBE, 0)),
            pl.BlockSpec((2, _BE, _D), lambda i: (0, i, 0)),
            pl.BlockSpec((_D, _D), lambda i: (0, 0)),
            pl.BlockSpec((1, _D), lambda i: (0, 0)),
            pl.BlockSpec((1, _D), lambda i: (0, 0)),
        ],
        out_specs=[
            pl.BlockSpec((_BE, _D), lambda i: (i, 0)),
            pl.BlockSpec((_BE, _D), lambda i: (i, 0)),
        ],
        out_shape=[
            jax.ShapeDtypeStruct((_EC, _D), jnp.float32),
            jax.ShapeDtypeStruct((_EC, _D), jnp.float32),
        ],
    )(edge2d, gpair, w1, g1, b1)


def _tc_node(node2d, aggp, deg, w, g2, b2):
    def body(x_ref, a_ref, d_ref, w_ref, g_ref, b_ref, o_ref):
        x = x_ref[...]
        agg = (a_ref[0] + a_ref[1]) / d_ref[...]
        h = jnp.dot(x, w_ref[:_D], preferred_element_type=jnp.float32)
        h = h + jnp.dot(agg, w_ref[_D:], preferred_element_type=jnp.float32)
        t = jnp.tanh(h)
        mu = jnp.mean(t, axis=-1, keepdims=True)
        var = jnp.mean((t - mu) ** 2, axis=-1, keepdims=True)
        o_ref[...] = x + (t - mu) / jnp.sqrt(var + _EPS) * g_ref[...] + b_ref[...]

    return pl.pallas_call(
        body,
        grid=(_N // _BN,),
        in_specs=[
            pl.BlockSpec((_BN, _D), lambda i: (i, 0)),
            pl.BlockSpec((2, _BN, _D), lambda i: (0, i, 0)),
            pl.BlockSpec((_BN, 1), lambda i: (i, 0)),
            pl.BlockSpec((2 * _D, _D), lambda i: (0, 0)),
            pl.BlockSpec((1, _D), lambda i: (0, 0)),
            pl.BlockSpec((1, _D), lambda i: (0, 0)),
        ],
        out_specs=pl.BlockSpec((_BN, _D), lambda i: (i, 0)),
        out_shape=jax.ShapeDtypeStruct((_N, _D), jnp.float32),
    )(node2d, aggp, deg, w, g2, b2)


def kernel(mesh_mesh_bond_embedding, mesh_node_embedding, edge_pairs,
           num_of_linked_nodes, W_n2e, g1, b1, W_e2n, g2, b2):
    node2d = mesh_node_embedding.reshape(_N, _D)
    edge2d = mesh_mesh_bond_embedding.reshape(_E, _D)
    src = edge_pairs[:, 0]
    dst = edge_pairs[:, 1]
    ipadn = _N + (jnp.arange(_EP - _E, dtype=jnp.int32) % (_NPAD - _N))
    srcp = jnp.concatenate([src, ipadn]).reshape(_C, _EC)
    dstp = jnp.concatenate([dst + _N, ipadn]).reshape(_C, _EC)
    gidx_all = jnp.concatenate([srcp, dstp], axis=1)
    zeros_hbm = jnp.zeros((_NPAD, _D), jnp.float32)

    proj = _tc_node_proj(node2d, W_n2e[_D:])
    table = proj.reshape(2 * _N, _D)
    w1 = W_n2e[:_D]
    g1r, b1r = g1.reshape(1, _D), b1.reshape(1, _D)

    gpairs = [_sc_gather(table, gidx_all, c).reshape(2, _EC, _D)
              for c in range(_C)]
    edge_outs, deltas = [], []
    for c in range(_C):
        edge_out_c, delta_c = _tc_edge(edge2d, gpairs[c], w1, g1r, b1r, c)
        edge_outs.append(edge_out_c)
        deltas.append(delta_c)
    aggp = _sc_segment_sum(deltas, srcp, zeros_hbm)

    node_out2 = _tc_node(node2d, aggp, num_of_linked_nodes, W_e2n,
                         g2.reshape(1, _D), b2.reshape(1, _D))
    edge_out2 = jnp.concatenate(edge_outs, axis=0)[:_E]
    return (edge_out2.reshape(1, _E, _D), node_out2.reshape(1, _N, _D))
